# baseline (speedup 1.0000x reference)
import functools

import numpy as np
import jax
import jax.numpy as jnp
from jax import lax
from jax.experimental import pallas as pl
from jax.experimental.pallas import tpu as pltpu
from jax.experimental.pallas import tpu_sc as plsc

D = 768
NC = 2
NS = 16
NW = NC * NS
LANES = 16
CH = 16


@functools.lru_cache(maxsize=None)
def _pe_table(seq_len: int, d: int):
    pos = np.arange(seq_len, dtype=np.float64).reshape(-1, 1)
    i = np.arange(0, d, 2, dtype=np.float64).reshape(1, -1)
    denom = np.power(10000.0, i / d)
    pe = np.zeros((seq_len, d), dtype=np.float32)
    pe[:, 0::2] = np.sin(pos / denom)
    pe[:, 1::2] = np.cos(pos / denom)
    return jax.device_put(pe)


@functools.lru_cache(maxsize=None)
def _build(batch: int, seq_len: int, vocab: int, d: int):
    tok = batch * seq_len
    assert seq_len % NW == 0
    ppw = seq_len // NW
    assert ppw % CH == 0
    npb = ppw // CH
    nch = npb * batch
    assert npb % 2 == 0 and batch % 2 == 0

    mesh = plsc.VectorSubcoreMesh(
        core_axis_name="c", subcore_axis_name="s",
        num_cores=NC, num_subcores=NS,
    )

    @functools.partial(
        pl.kernel,
        out_type=jax.ShapeDtypeStruct((tok, d), jnp.float32),
        mesh=mesh,
        scratch_types=[
            pltpu.VMEM((batch, ppw), jnp.int32),
            pltpu.VMEM((CH, d), jnp.float32),
            pltpu.VMEM((CH, d), jnp.float32),
            pltpu.VMEM((CH, d), jnp.float32),
            pltpu.VMEM((CH, d), jnp.float32),
            pltpu.VMEM((CH, d), jnp.float32),
            pltpu.VMEM((CH, d), jnp.float32),
            pltpu.SemaphoreType.DMA,
            pltpu.SemaphoreType.DMA,
            pltpu.SemaphoreType.DMA,
            pltpu.SemaphoreType.DMA,
            pltpu.SemaphoreType.DMA,
            pltpu.SemaphoreType.DMA,
        ],
    )
    def emb_kernel(ids_hbm, table_hbm, pe_hbm, out_hbm,
                   idx_all, in0, in1, out0, out1, pe0, pe1,
                   g0, g1, p0, p1, s0, s1):
        wid = lax.axis_index("s") * NC + lax.axis_index("c")
        pos0 = wid * ppw

        gbufs = ((in0, out0, g0, s0), (in1, out1, g1, s1))
        pebufs = ((pe0, p0), (pe1, p1))

        def gather_cp(p, bat, inb, gs):
            idx = idx_all.at[bat, pl.ds(p * CH, CH)]
            return pltpu.make_async_copy(table_hbm.at[idx], inb, gs)

        def pe_cp(p, peb, ps):
            return pltpu.make_async_copy(
                pe_hbm.at[pl.ds(pos0 + p * CH, CH)], peb, ps)

        def store_cp(p, bat, outb, ss):
            row0 = bat * seq_len + pos0 + p * CH
            return pltpu.make_async_copy(
                outb, out_hbm.at[pl.ds(row0, CH)], ss)

        for bat in range(batch):
            pltpu.sync_copy(ids_hbm.at[pl.ds(bat * seq_len + pos0, ppw)],
                            idx_all.at[bat])
        for par in range(2):
            inb, outb, gs, ss = gbufs[par]
            gather_cp(0, par, inb, gs).start()
        pe_cp(0, pe0, p0).start()

        def outer(i, carry):
            for pp in range(2):
                p = i * 2 + pp
                peb, ps = pebufs[pp]
                pe_cp(p, peb, ps).wait()

                @pl.when(p + 1 < npb)
                def _():
                    pe_cp(p + 1, pebufs[1 - pp][0], pebufs[1 - pp][1]).start()

                for bat in range(batch):
                    c = p * batch + bat
                    par = bat % 2
                    inb, outb, gs, ss = gbufs[par]
                    gather_cp(p, bat, inb, gs).wait()

                    @pl.when(c >= 2)
                    def _():
                        pm2 = lax.div(c - 2, batch)
                        bm2 = lax.rem(c - 2, batch)
                        store_cp(pm2, bm2, outb, ss).wait()

                    def add_row(r, rcarry):
                        for k in range(d // LANES):
                            sl = pl.ds(k * LANES, LANES)
                            outb[r, sl] = inb[r, sl] + peb[r, sl]
                        return rcarry

                    lax.fori_loop(0, CH, add_row, 0)
                    store_cp(p, bat, outb, ss).start()

                    @pl.when(c + 2 < nch)
                    def _():
                        pn = lax.div(c + 2, batch)
                        bn = lax.rem(c + 2, batch)
                        gather_cp(pn, bn, inb, gs).start()
            return carry

        lax.fori_loop(0, npb // 2, outer, 0)

        for bat in (batch - 2, batch - 1):
            par = bat % 2
            inb, outb, gs, ss = gbufs[par]
            store_cp(npb - 1, bat, outb, ss).wait()

    return emb_kernel


def kernel(token_ids, table):
    b, s = token_ids.shape
    vocab, d = table.shape
    ids = token_ids.astype(jnp.int32).reshape(-1)
    pe = _pe_table(s, d)
    out = _build(b, s, vocab, d)(ids, table, pe)
    return out.reshape(b, s, d)
